# baseline (speedup 1.0000x reference)
import functools
import jax
import jax.numpy as jnp
from jax import lax
from jax.experimental import pallas as pl
from jax.experimental.pallas import tpu as pltpu, tpu_sc as plsc

_T = 100
_B = 4096
_F = 128
_L = 16

_info = plsc.get_sparse_core_info()
_NC, _NS = _info.num_cores, _info.num_subcores
_NW = _NC * _NS
_RPW = _B // _NW


def _spike_cols(xbuf, row, c):
    xv = xbuf[row, pl.ds(c * _L, _L)]
    st = ((1.0 - xv) * _T).astype(jnp.int32)
    valid = st < _T
    idx_r = jnp.minimum(st, _T - 1)
    return idx_r, valid


def _scatter_row(buf, xbuf, row, value):
    vals = jnp.full((_L,), value, jnp.float32)
    for c in range(_F // _L):
        idx_r, valid = _spike_cols(xbuf, row, c)
        idx_c = jnp.arange(_L, dtype=jnp.int32) + c * _L
        plsc.store_scatter(buf, [idx_r, idx_c], vals, mask=valid)


def _sc_kernel(x_hbm, out_hbm, xbuf, buf0, buf1, sem0, sem1):
    wid = lax.axis_index("s") * _NC + lax.axis_index("c")
    base = wid * _RPW
    bufs = (buf0, buf1)
    sems = (sem0, sem1)

    pltpu.sync_copy(x_hbm.at[pl.ds(base, _RPW)], xbuf)

    def _zero(t, _):
        z = jnp.zeros((_L,), jnp.float32)
        for c in range(_F // _L):
            buf0[t, pl.ds(c * _L, _L)] = z
            buf1[t, pl.ds(c * _L, _L)] = z
        return 0

    lax.fori_loop(0, _T, _zero, 0)

    for b in range(2):
        _scatter_row(bufs[b], xbuf, b, 1.0)
        pltpu.async_copy(bufs[b], out_hbm.at[base + b], sems[b])

    def _step(i, _):
        r0 = 2 * i
        for b in range(2):
            row = r0 + b
            pltpu.make_async_copy(bufs[b], out_hbm.at[base], sems[b]).wait()
            _scatter_row(bufs[b], xbuf, row - 2, 0.0)
            _scatter_row(bufs[b], xbuf, row, 1.0)
            pltpu.async_copy(bufs[b], out_hbm.at[base + row], sems[b])
        return 0

    lax.fori_loop(1, _RPW // 2, _step, 0)

    for b in range(2):
        pltpu.make_async_copy(bufs[b], out_hbm.at[base], sems[b]).wait()


def kernel(x):
    mesh = plsc.VectorSubcoreMesh(core_axis_name="c", subcore_axis_name="s")
    run = functools.partial(
        pl.kernel,
        out_type=jax.ShapeDtypeStruct((_B, _T, _F), jnp.float32),
        mesh=mesh,
        scratch_types=[
            pltpu.VMEM((_RPW, _F), jnp.float32),
            pltpu.VMEM((_T, _F), jnp.float32),
            pltpu.VMEM((_T, _F), jnp.float32),
            pltpu.SemaphoreType.DMA,
            pltpu.SemaphoreType.DMA,
        ],
        compiler_params=pltpu.CompilerParams(needs_layout_passes=False),
    )(_sc_kernel)
    return run(x)
